# baseline (speedup 1.0000x reference)
import functools

import jax
import jax.numpy as jnp
from jax import lax
from jax.experimental import pallas as pl
from jax.experimental.pallas import tpu as pltpu

_F32 = jnp.float32
_BF16 = jnp.bfloat16
_EPS = 1e-9
_NB = 8


def _leaky(x):
    return jnp.maximum(x, 0.2 * x)


def _epsx(u, vrow):
    return jnp.reshape(_EPS * jnp.exp(_leaky(jnp.max(u) + jnp.max(vrow))), (1, 1))


def _aug(s, n, dt=_F32):
    return jnp.concatenate(
        [s, jnp.ones((n, 1), _F32), jnp.zeros((n, 63), _F32)],
        axis=1).astype(dt)


def _vrow(a, s):
    return lax.dot_general(a, s, (((0,), (1,)), ((), ())),
                           preferred_element_type=_F32)


def _att_block(ex_ref, u_blk, vrow_ref, saug_ref, n_ref, dt=_F32):
    z = u_blk + vrow_ref[...]
    w = (n_ref[...] * jnp.exp(_leaky(z))).astype(dt)
    f = jnp.dot(w, saug_ref[...], preferred_element_type=_F32)
    out = f[:, :64] / (f[:, 64:65] + ex_ref[0, 0])
    return w, out


def _proj_sq(x, W, a_src, a_dst, saug, us, vr, ex, n):
    s = jnp.dot(x, W[...], preferred_element_type=_F32)
    u = jnp.dot(s, a_src[...], preferred_element_type=_F32)
    v = _vrow(a_dst[...], s)
    saug[...] = _aug(s, n)
    us[...] = u
    vr[...] = v
    ex[...] = _epsx(u, v)


def _proj_ns(xs, xt, Ws, Wt, a_s, a_t, saug, us, vr, ex, ns_, nt_,
             taug=None, dt=_F32):
    s = jnp.dot(xs, Ws[...], preferred_element_type=_F32)
    t = jnp.dot(xt, Wt[...], preferred_element_type=_F32)
    u = jnp.dot(t, a_t[...], preferred_element_type=_F32)
    v = _vrow(a_s[...], s)
    saug[...] = _aug(s, ns_, dt)
    if taug is not None:
        taug[...] = _aug(t, nt_, dt)
    us[...] = u
    vr[...] = v
    ex[...] = _epsx(u, v)


def _mono_body(x0_ref, x1_ref, x2_ref,
               Wa01_ref, aas1_ref, aad1_ref,
               Wsb11_ref, Wtb11_ref, ab1s1_ref, ab1t1_ref,
               Wsb21_ref, Wtb21_ref, ab2s1_ref, ab2t1_ref,
               Wa02_ref, aas2_ref, aad2_ref,
               Wsb12_ref, Wtb12_ref, ab1s2_ref, ab1t2_ref,
               Wd1_ref, bd1_ref, Wd2_ref, bd2_ref,
               a0_ref, b1_ref, b2_ref, o_ref,
               s0aug, sb1aug, sb2aug, tb1aug,
               u0s, ub1s, ub2s, v0r, vb1r, vb2r,
               ex0, exb1, exb2, acc,
               h00s, m0f1s, m1f0s, m1f2s, *, bi0, bi2):
    i = pl.program_id(0)
    nb = _NB

    @pl.when(i == 0)
    def _prep1():
        x0 = x0_ref[...]
        x1 = x1_ref[...]
        x2 = x2_ref[...]
        n0, n1, n2 = x0.shape[0], x1.shape[0], x2.shape[0]
        _proj_sq(x0, Wa01_ref, aas1_ref, aad1_ref, s0aug, u0s, v0r, ex0, n0)
        _proj_ns(x1, x0, Wsb11_ref, Wtb11_ref, ab1s1_ref, ab1t1_ref,
                 sb1aug, ub1s, vb1r, exb1, n1, n0, taug=tb1aug, dt=_BF16)
        _proj_ns(x2, x1, Wsb21_ref, Wtb21_ref, ab2s1_ref, ab2t1_ref,
                 sb2aug, ub2s, vb2r, exb2, n2, n1)

    @pl.when((i >= 1) & (i <= nb))
    def _l1():
        k = i - 1
        _, h00 = _att_block(ex0, u0s[pl.ds(k * bi0, bi0), :], v0r, s0aug, a0_ref)
        h00s[pl.ds(k * bi0, bi0), :] = h00
        w1, m0f1 = _att_block(exb1, ub1s[pl.ds(k * bi0, bi0), :], vb1r,
                              sb1aug, b1_ref, dt=_BF16)
        m0f1s[pl.ds(k * bi0, bi0), :] = m0f1
        _, m1f2 = _att_block(exb2, ub2s[pl.ds(k * bi2, bi2), :], vb2r,
                             sb2aug, b2_ref)
        m1f2s[pl.ds(k * bi2, bi2), :] = m1f2
        dn = (((0,), (0,)), ((), ()))
        part = lax.dot_general(w1, tb1aug[pl.ds(k * bi0, bi0), :], dn,
                               preferred_element_type=_F32)

        @pl.when(i == 1)
        def _():
            acc[...] = part

        @pl.when(i > 1)
        def _():
            acc[...] += part

        @pl.when(i == nb)
        def _():
            a = acc[...]
            m1f0s[...] = a[:, :64] / (a[:, 64:65] + exb1[0, 0])

    @pl.when(i == nb + 1)
    def _prep2():
        x0l1 = jnp.maximum(h00s[...] + m0f1s[...], 0.0)
        x1l1 = jnp.maximum(m1f0s[...] + m1f2s[...], 0.0)
        n0, n1 = x0l1.shape[0], x1l1.shape[0]
        _proj_sq(x0l1, Wa02_ref, aas2_ref, aad2_ref, s0aug, u0s, v0r, ex0, n0)
        _proj_ns(x1l1, x0l1, Wsb12_ref, Wtb12_ref, ab1s2_ref, ab1t2_ref,
                 sb1aug, ub1s, vb1r, exb1, n1, n0, dt=_BF16)

    @pl.when(i >= nb + 2)
    def _l2():
        k = i - nb - 2
        _, h00 = _att_block(ex0, u0s[pl.ds(k * bi0, bi0), :], v0r, s0aug, a0_ref)
        _, m0f1 = _att_block(exb1, ub1s[pl.ds(k * bi0, bi0), :], vb1r,
                             sb1aug, b1_ref)
        x0e = jnp.maximum(h00 + m0f1, 0.0)
        h = jnp.maximum(
            jnp.dot(x0e, Wd1_ref[...], preferred_element_type=_F32)
            + bd1_ref[...], 0.0)
        o_ref[...] = (jnp.dot(h, Wd2_ref[...], preferred_element_type=_F32)
                      + bd2_ref[...])


def _forward_fused(x0, x1, x2, a0, b1, b2, Wa01, aas1, aad1,
                   Wsb11, Wtb11, ab1s1, ab1t1, Wsb21, Wtb21, ab2s1, ab2t1,
                   Wa02, aas2, aad2, Wsb12, Wtb12, ab1s2, ab1t2,
                   Wd1, bd1, Wd2, bd2):
    n0 = a0.shape[0]
    n1, n2 = b2.shape
    d_out = Wd2.shape[1]
    nb = _NB
    bi0 = n0 // nb
    bi2 = n1 // nb
    c = lambda blk: pl.BlockSpec(blk, lambda i: (0, 0))

    def _two_phase(i):
        return (jnp.clip(jnp.where(i < nb + 1, i - 1, i - nb - 2), 0, nb - 1), 0)

    def _one_phase(i):
        return (jnp.clip(i - 1, 0, nb - 1), 0)

    def _out_map(i):
        return (jnp.clip(i - nb - 2, 0, nb - 1), 0)

    return pl.pallas_call(
        functools.partial(_mono_body, bi0=bi0, bi2=bi2),
        grid=(2 * nb + 2,),
        in_specs=[
            c((n0, 128)), c((n1, 128)), c((n2, 128)),
            c((128, 64)), c((64, 1)), c((64, 1)),
            c((128, 64)), c((128, 64)), c((64, 1)), c((64, 1)),
            c((128, 64)), c((128, 64)), c((64, 1)), c((64, 1)),
            c((64, 64)), c((64, 1)), c((64, 1)),
            c((64, 64)), c((64, 64)), c((64, 1)), c((64, 1)),
            c((64, 64)), c((1, 64)), c((64, d_out)), c((1, d_out)),
            pl.BlockSpec((bi0, n0), _two_phase),
            pl.BlockSpec((bi0, n1), _two_phase),
            pl.BlockSpec((bi2, n2), _one_phase),
        ],
        out_specs=pl.BlockSpec((bi0, d_out), _out_map),
        out_shape=jax.ShapeDtypeStruct((n0, d_out), _F32),
        scratch_shapes=[
            pltpu.VMEM((n0, 128), _F32), pltpu.VMEM((n1, 128), _BF16),
            pltpu.VMEM((n2, 128), _F32), pltpu.VMEM((n0, 128), _BF16),
            pltpu.VMEM((n0, 1), _F32), pltpu.VMEM((n0, 1), _F32),
            pltpu.VMEM((n1, 1), _F32),
            pltpu.VMEM((1, n0), _F32), pltpu.VMEM((1, n1), _F32),
            pltpu.VMEM((1, n2), _F32),
            pltpu.VMEM((1, 1), _F32), pltpu.VMEM((1, 1), _F32),
            pltpu.VMEM((1, 1), _F32), pltpu.VMEM((n1, 128), _F32),
            pltpu.VMEM((n0, 64), _F32), pltpu.VMEM((n0, 64), _F32),
            pltpu.VMEM((n1, 64), _F32), pltpu.VMEM((n1, 64), _F32),
        ],
    )(x0, x1, x2, Wa01, aas1, aad1, Wsb11, Wtb11, ab1s1, ab1t1,
      Wsb21, Wtb21, ab2s1, ab2t1, Wa02, aas2, aad2,
      Wsb12, Wtb12, ab1s2, ab1t2, Wd1, bd1, Wd2, bd2, a0, b1, b2)


def kernel(x_0, x_1, x_2, a0, a1, coa2, b1, b2,
           W_a0_1, asrc_a0_1, adst_a0_1,
           Ws_b1_1, Wt_b1_1, as_b1_1, at_b1_1,
           Ws_b2_1, Wt_b2_1, as_b2_1, at_b2_1,
           W_a0_2, asrc_a0_2, adst_a0_2,
           W_a1_2, asrc_a1_2, adst_a1_2,
           W_co2_2, asrc_co2_2, adst_co2_2,
           Ws_b1_2, Wt_b1_2, as_b1_2, at_b1_2,
           Ws_b2_2, Wt_b2_2, as_b2_2, at_b2_2,
           Wd1, bd1, Wd2, bd2):
    x0 = x_0[0]
    x1 = x_1[0]
    x2 = x_2[0]
    col = lambda a: jnp.reshape(a, (-1, 1))
    row = lambda a: jnp.reshape(a, (1, -1))

    recon = _forward_fused(
        x0, x1, x2, a0, b1, b2,
        W_a0_1, col(asrc_a0_1), col(adst_a0_1),
        Ws_b1_1, Wt_b1_1, col(as_b1_1), col(at_b1_1),
        Ws_b2_1, Wt_b2_1, col(as_b2_1), col(at_b2_1),
        W_a0_2, col(asrc_a0_2), col(adst_a0_2),
        Ws_b1_2, Wt_b1_2, col(as_b1_2), col(at_b1_2),
        Wd1, row(bd1), Wd2, row(bd2))
    return recon[None, :, :]


# speedup vs baseline: 1.0262x; 1.0262x over previous
import functools

import jax
import jax.numpy as jnp
from jax import lax
from jax.experimental import pallas as pl
from jax.experimental.pallas import tpu as pltpu

_F32 = jnp.float32
_EPS = 1e-9
_LOG2E = 1.4426950408889634
_NB = 8


def _leaky(x):
    return jnp.maximum(x, 0.2 * x)


def _epsx(u, vrow):
    return jnp.reshape(
        _EPS * jnp.exp2(_leaky(jnp.max(u) + jnp.max(vrow))), (1, 1))


def _aug(s, n):
    return jnp.concatenate(
        [s, jnp.ones((n, 1), _F32), jnp.zeros((n, 63), _F32)], axis=1)


def _vrow(a, s):
    return lax.dot_general(a, s, (((0,), (1,)), ((), ())),
                           preferred_element_type=_F32)


def _att_block(ex_ref, u_blk, vrow_ref, saug_ref, n_ref):
    z = u_blk + vrow_ref[...]
    w = n_ref[...] * jnp.exp2(_leaky(z))
    f = jnp.dot(w, saug_ref[...], preferred_element_type=_F32)
    out = f[:, :64] / (f[:, 64:65] + ex_ref[0, 0])
    return w, out


def _proj_sq(x, W, a_src, a_dst, saug, us, vr, ex, n):
    s = jnp.dot(x, W[...], preferred_element_type=_F32)
    u = jnp.dot(s, a_src[...], preferred_element_type=_F32) * _LOG2E
    v = _vrow(a_dst[...], s) * _LOG2E
    saug[...] = _aug(s, n)
    us[...] = u
    vr[...] = v
    ex[...] = _epsx(u, v)


def _proj_ns(xs, xt, Ws, Wt, a_s, a_t, saug, us, vr, ex, ns_, nt_, taug=None):
    s = jnp.dot(xs, Ws[...], preferred_element_type=_F32)
    t = jnp.dot(xt, Wt[...], preferred_element_type=_F32)
    u = jnp.dot(t, a_t[...], preferred_element_type=_F32) * _LOG2E
    v = _vrow(a_s[...], s) * _LOG2E
    saug[...] = _aug(s, ns_)
    if taug is not None:
        taug[...] = _aug(t, nt_)
    us[...] = u
    vr[...] = v
    ex[...] = _epsx(u, v)


def _mono_body(x0_ref, x1_ref, x2_ref,
               Wa01_ref, aas1_ref, aad1_ref,
               Wsb11_ref, Wtb11_ref, ab1s1_ref, ab1t1_ref,
               Wsb21_ref, Wtb21_ref, ab2s1_ref, ab2t1_ref,
               Wa02_ref, aas2_ref, aad2_ref,
               Wsb12_ref, Wtb12_ref, ab1s2_ref, ab1t2_ref,
               Wd1_ref, bd1_ref, Wd2_ref, bd2_ref,
               a0_ref, b1_ref, b2_ref, o_ref,
               s0aug, sb1aug, sb2aug, tb1aug,
               u0s, ub1s, ub2s, v0r, vb1r, vb2r,
               ex0, exb1, exb2, acc,
               h00s, m0f1s, m1f0s, m1f2s, *, bi0, bi2):
    i = pl.program_id(0)
    nb = _NB

    @pl.when(i == 0)
    def _prep1():
        x0 = x0_ref[...]
        x1 = x1_ref[...]
        x2 = x2_ref[...]
        n0, n1, n2 = x0.shape[0], x1.shape[0], x2.shape[0]
        _proj_sq(x0, Wa01_ref, aas1_ref, aad1_ref, s0aug, u0s, v0r, ex0, n0)
        _proj_ns(x1, x0, Wsb11_ref, Wtb11_ref, ab1s1_ref, ab1t1_ref,
                 sb1aug, ub1s, vb1r, exb1, n1, n0, taug=tb1aug)
        _proj_ns(x2, x1, Wsb21_ref, Wtb21_ref, ab2s1_ref, ab2t1_ref,
                 sb2aug, ub2s, vb2r, exb2, n2, n1)

    @pl.when((i >= 1) & (i <= nb))
    def _l1():
        k = i - 1
        _, h00 = _att_block(ex0, u0s[pl.ds(k * bi0, bi0), :], v0r, s0aug, a0_ref)
        h00s[pl.ds(k * bi0, bi0), :] = h00
        w1, m0f1 = _att_block(exb1, ub1s[pl.ds(k * bi0, bi0), :], vb1r,
                              sb1aug, b1_ref)
        m0f1s[pl.ds(k * bi0, bi0), :] = m0f1
        _, m1f2 = _att_block(exb2, ub2s[pl.ds(k * bi2, bi2), :], vb2r,
                             sb2aug, b2_ref)
        m1f2s[pl.ds(k * bi2, bi2), :] = m1f2
        dn = (((0,), (0,)), ((), ()))
        part = lax.dot_general(w1, tb1aug[pl.ds(k * bi0, bi0), :], dn,
                               preferred_element_type=_F32)

        @pl.when(i == 1)
        def _():
            acc[...] = part

        @pl.when(i > 1)
        def _():
            acc[...] += part

        @pl.when(i == nb)
        def _():
            a = acc[...]
            m1f0s[...] = a[:, :64] / (a[:, 64:65] + exb1[0, 0])

    @pl.when(i == nb + 1)
    def _prep2():
        x0l1 = jnp.maximum(h00s[...] + m0f1s[...], 0.0)
        x1l1 = jnp.maximum(m1f0s[...] + m1f2s[...], 0.0)
        n0, n1 = x0l1.shape[0], x1l1.shape[0]
        _proj_sq(x0l1, Wa02_ref, aas2_ref, aad2_ref, s0aug, u0s, v0r, ex0, n0)
        _proj_ns(x1l1, x0l1, Wsb12_ref, Wtb12_ref, ab1s2_ref, ab1t2_ref,
                 sb1aug, ub1s, vb1r, exb1, n1, n0)

    @pl.when(i >= nb + 2)
    def _l2():
        k = i - nb - 2
        _, h00 = _att_block(ex0, u0s[pl.ds(k * bi0, bi0), :], v0r, s0aug, a0_ref)
        _, m0f1 = _att_block(exb1, ub1s[pl.ds(k * bi0, bi0), :], vb1r,
                             sb1aug, b1_ref)
        x0e = jnp.maximum(h00 + m0f1, 0.0)
        h = jnp.maximum(
            jnp.dot(x0e, Wd1_ref[...], preferred_element_type=_F32)
            + bd1_ref[...], 0.0)
        o_ref[...] = (jnp.dot(h, Wd2_ref[...], preferred_element_type=_F32)
                      + bd2_ref[...])


def _forward_fused(x0, x1, x2, a0, b1, b2, Wa01, aas1, aad1,
                   Wsb11, Wtb11, ab1s1, ab1t1, Wsb21, Wtb21, ab2s1, ab2t1,
                   Wa02, aas2, aad2, Wsb12, Wtb12, ab1s2, ab1t2,
                   Wd1, bd1, Wd2, bd2):
    n0 = a0.shape[0]
    n1, n2 = b2.shape
    d_out = Wd2.shape[1]
    nb = _NB
    bi0 = n0 // nb
    bi2 = n1 // nb
    c = lambda blk: pl.BlockSpec(blk, lambda i: (0, 0))

    def _two_phase(i):
        return (jnp.clip(jnp.where(i < nb + 1, i - 1, i - nb - 2), 0, nb - 1), 0)

    def _one_phase(i):
        return (jnp.clip(i - 1, 0, nb - 1), 0)

    def _out_map(i):
        return (jnp.clip(i - nb - 2, 0, nb - 1), 0)

    return pl.pallas_call(
        functools.partial(_mono_body, bi0=bi0, bi2=bi2),
        grid=(2 * nb + 2,),
        in_specs=[
            c((n0, 128)), c((n1, 128)), c((n2, 128)),
            c((128, 64)), c((64, 1)), c((64, 1)),
            c((128, 64)), c((128, 64)), c((64, 1)), c((64, 1)),
            c((128, 64)), c((128, 64)), c((64, 1)), c((64, 1)),
            c((64, 64)), c((64, 1)), c((64, 1)),
            c((64, 64)), c((64, 64)), c((64, 1)), c((64, 1)),
            c((64, 64)), c((1, 64)), c((64, d_out)), c((1, d_out)),
            pl.BlockSpec((bi0, n0), _two_phase),
            pl.BlockSpec((bi0, n1), _two_phase),
            pl.BlockSpec((bi2, n2), _one_phase),
        ],
        out_specs=pl.BlockSpec((bi0, d_out), _out_map),
        out_shape=jax.ShapeDtypeStruct((n0, d_out), _F32),
        scratch_shapes=[
            pltpu.VMEM((n0, 128), _F32), pltpu.VMEM((n1, 128), _F32),
            pltpu.VMEM((n2, 128), _F32), pltpu.VMEM((n0, 128), _F32),
            pltpu.VMEM((n0, 1), _F32), pltpu.VMEM((n0, 1), _F32),
            pltpu.VMEM((n1, 1), _F32),
            pltpu.VMEM((1, n0), _F32), pltpu.VMEM((1, n1), _F32),
            pltpu.VMEM((1, n2), _F32),
            pltpu.VMEM((1, 1), _F32), pltpu.VMEM((1, 1), _F32),
            pltpu.VMEM((1, 1), _F32), pltpu.VMEM((n1, 128), _F32),
            pltpu.VMEM((n0, 64), _F32), pltpu.VMEM((n0, 64), _F32),
            pltpu.VMEM((n1, 64), _F32), pltpu.VMEM((n1, 64), _F32),
        ],
    )(x0, x1, x2, Wa01, aas1, aad1, Wsb11, Wtb11, ab1s1, ab1t1,
      Wsb21, Wtb21, ab2s1, ab2t1, Wa02, aas2, aad2,
      Wsb12, Wtb12, ab1s2, ab1t2, Wd1, bd1, Wd2, bd2, a0, b1, b2)


def kernel(x_0, x_1, x_2, a0, a1, coa2, b1, b2,
           W_a0_1, asrc_a0_1, adst_a0_1,
           Ws_b1_1, Wt_b1_1, as_b1_1, at_b1_1,
           Ws_b2_1, Wt_b2_1, as_b2_1, at_b2_1,
           W_a0_2, asrc_a0_2, adst_a0_2,
           W_a1_2, asrc_a1_2, adst_a1_2,
           W_co2_2, asrc_co2_2, adst_co2_2,
           Ws_b1_2, Wt_b1_2, as_b1_2, at_b1_2,
           Ws_b2_2, Wt_b2_2, as_b2_2, at_b2_2,
           Wd1, bd1, Wd2, bd2):
    x0 = x_0[0]
    x1 = x_1[0]
    x2 = x_2[0]
    col = lambda a: jnp.reshape(a, (-1, 1))
    row = lambda a: jnp.reshape(a, (1, -1))

    recon = _forward_fused(
        x0, x1, x2, a0, b1, b2,
        W_a0_1, col(asrc_a0_1), col(adst_a0_1),
        Ws_b1_1, Wt_b1_1, col(as_b1_1), col(at_b1_1),
        Ws_b2_1, Wt_b2_1, col(as_b2_1), col(at_b2_1),
        W_a0_2, col(asrc_a0_2), col(adst_a0_2),
        Ws_b1_2, Wt_b1_2, col(as_b1_2), col(at_b1_2),
        Wd1, row(bd1), Wd2, row(bd2))
    return recon[None, :, :]
